# baseline (speedup 1.0000x reference)
import dataclasses
import functools

import jax
import jax.numpy as jnp
from jax import lax
from jax.experimental import pallas as pl
from jax.experimental.pallas import tpu as pltpu
from jax.experimental.pallas import tpu_sc as plsc

N = 10000
E = 320000
D = 128
H = 8
DK = 16
NT = 32
RNG = 320
NP = NT * RNG
CAP = 11264
ECHUNK = 1280
CHUNK = 128

_mesh = plsc.VectorSubcoreMesh(core_axis_name="c", subcore_axis_name="s")

_sc_params = pltpu.CompilerParams()
if "needs_layout_passes" in pltpu.CompilerParams.__dataclass_fields__:
    _sc_params = dataclasses.replace(_sc_params, needs_layout_passes=False)


def _wid():
    return lax.axis_index("s") * 2 + lax.axis_index("c")


def _bucket(src, dst):
    @functools.partial(
        pl.kernel,
        out_type=(
            jax.ShapeDtypeStruct((NT, CAP), jnp.int32),
            jax.ShapeDtypeStruct((NT, CAP), jnp.int32),
            jax.ShapeDtypeStruct((NT, 128), jnp.int32),
        ),
        mesh=_mesh,
        scratch_types=[
            pltpu.VMEM((ECHUNK,), jnp.int32),
            pltpu.VMEM((ECHUNK,), jnp.int32),
            pltpu.VMEM((CAP,), jnp.int32),
            pltpu.VMEM((CAP,), jnp.int32),
            pltpu.VMEM((128,), jnp.int32),
            pltpu.SemaphoreType.DMA,
        ],
        compiler_params=_sc_params,
    )
    def k(src_hbm, dst_hbm, src_out, dst_out, cnt_out, sbuf, dbuf, ssel, dsel,
          cntv, sem):
        t = _wid()
        lo = t * RNG
        hi = lo + RNG
        zero = jnp.zeros((16,), jnp.int32)

        hivec = jnp.full((16,), hi, jnp.int32)

        @pl.loop(0, CAP // 16)
        def _(i):
            ssel[pl.ds(i * 16, 16)] = zero
            dsel[pl.ds(i * 16, 16)] = hivec

        def chunk_body(c, off):
            pltpu.sync_copy(src_hbm.at[pl.ds(c * ECHUNK, ECHUNK)], sbuf)
            pltpu.sync_copy(dst_hbm.at[pl.ds(c * ECHUNK, ECHUNK)], dbuf)

            def vec_body(j, off):
                dvec = dbuf[pl.ds(j * 16, 16)]
                svec = sbuf[pl.ds(j * 16, 16)]
                m = (dvec >= lo) & (dvec < hi)
                off = jnp.minimum(off, CAP - 16)
                plsc.store_compressed(ssel.at[pl.ds(off, 16)], svec, mask=m)
                plsc.store_compressed(dsel.at[pl.ds(off, 16)], dvec, mask=m)
                return off + jnp.sum(m.astype(jnp.int32))

            return lax.fori_loop(0, ECHUNK // 16, vec_body, off)

        cnt = lax.fori_loop(0, E // ECHUNK, chunk_body, jnp.int32(0))
        pltpu.sync_copy(ssel, src_out.at[t])
        pltpu.sync_copy(dsel, dst_out.at[t])
        cv = jnp.full((16,), cnt, jnp.int32)

        @pl.loop(0, 8)
        def _(i):
            cntv[pl.ds(i * 16, 16)] = cv

        pltpu.sync_copy(cntv, cnt_out.at[t])

    return k(src, dst)


def _edge(q, kk, v, src_s, dst_s, counts):
    @functools.partial(
        pl.kernel,
        out_type=(
            jax.ShapeDtypeStruct((NP, D), jnp.float32),
            jax.ShapeDtypeStruct((NP * 16,), jnp.float32),
        ),
        mesh=_mesh,
        scratch_types=(
            [pltpu.VMEM(((RNG + 8) * DK,), jnp.float32) for _ in range(H)] + [
                pltpu.VMEM(((RNG + 8) * 16,), jnp.float32),
                pltpu.VMEM((CHUNK,), jnp.int32),
                pltpu.VMEM((CHUNK,), jnp.int32),
                pltpu.VMEM((CHUNK // 2, D), jnp.float32),
                pltpu.VMEM((CHUNK // 2, D), jnp.float32),
                pltpu.VMEM((CHUNK // 2, D), jnp.float32),
                pltpu.SemaphoreType.DMA,
            ]
        ),
        compiler_params=_sc_params,
    )
    def k(q_hbm, k_hbm, v_hbm, src_hbm, dst_hbm, cnt_hbm, wv_out, z_out,
          a0, a1, a2, a3, a4, a5, a6, a7, z_acc, sidx, didx, kbuf, qbuf,
          vbuf, sem):
        acc = (a0, a1, a2, a3, a4, a5, a6, a7)
        t = _wid()
        lo = t * RNG
        zf = jnp.zeros((16,), jnp.float32)
        lane = lax.iota(jnp.int32, 16)
        x8 = lane ^ 8
        x4 = lane ^ 4
        x2 = lane ^ 2
        x1 = lane ^ 1

        @pl.loop(0, RNG + 8)
        def _(r):
            for hh in range(H):
                acc[hh][pl.ds(r * 16, 16)] = zf
            z_acc[pl.ds(r * 16, 16)] = zf

        pltpu.sync_copy(cnt_hbm.at[t], didx)
        n = didx[pl.ds(0, 16)][0]
        nch = (n + (CHUNK - 1)) // CHUNK

        @pl.loop(0, nch)
        def _(c):
            base = c * CHUNK
            pltpu.sync_copy(src_hbm.at[t, pl.ds(base, CHUNK)], sidx)
            pltpu.sync_copy(dst_hbm.at[t, pl.ds(base, CHUNK)], didx)
            for half in range(2):
                hb = half * (CHUNK // 2)
                si = sidx.at[pl.ds(hb, CHUNK // 2)]
                di = didx.at[pl.ds(hb, CHUNK // 2)]
                cp1 = pltpu.async_copy(k_hbm.at[si], kbuf, sem)
                cp2 = pltpu.async_copy(q_hbm.at[di], qbuf, sem)
                cp3 = pltpu.async_copy(v_hbm.at[si], vbuf, sem)
                cp1.wait()
                cp2.wait()
                cp3.wait()

                @pl.loop(0, CHUNK // 32)
                def _(g):
                    dlvec = didx[pl.ds(hb + g * 16, 16)] - lo
                    for jj in range(16):
                        dl = dlvec[jj]
                        row = g * 16 + jj
                        zrow = zf
                        for hh in range(H):
                            kv = kbuf[row, pl.ds(hh * DK, DK)]
                            qv = qbuf[row, pl.ds(hh * DK, DK)]
                            p = kv * qv
                            for x in (x8, x4, x2, x1):
                                p = p + p.at[x].get(
                                    mode="promise_in_bounds")
                            sc = jnp.exp(jnp.clip(p * 0.25, -5.0, 5.0))
                            wv = vbuf[row, pl.ds(hh * DK, DK)] * sc
                            plsc.addupdate(acc[hh].at[pl.ds(dl * 16, 16)],
                                           wv)
                            zrow = zrow + jnp.where(lane == hh, sc, 0.0)
                        plsc.addupdate(z_acc.at[pl.ds(dl * 16, 16)], zrow)

        @pl.loop(0, 64)
        def _(r):
            for buf, off in ((kbuf, 0), (qbuf, 64), (vbuf, 128)):
                for hh in range(H):
                    buf[r, pl.ds(hh * DK, DK)] = acc[hh][
                        pl.ds((off + r) * 16, 16)]

        pltpu.sync_copy(kbuf, wv_out.at[pl.ds(lo, 64)])
        pltpu.sync_copy(qbuf, wv_out.at[pl.ds(lo + 64, 64)])
        pltpu.sync_copy(vbuf, wv_out.at[pl.ds(lo + 128, 64)])

        @pl.loop(0, 64)
        def _(r):
            for buf, off in ((kbuf, 192), (qbuf, 256)):
                for hh in range(H):
                    buf[r, pl.ds(hh * DK, DK)] = acc[hh][
                        pl.ds((off + r) * 16, 16)]

        pltpu.sync_copy(kbuf, wv_out.at[pl.ds(lo + 192, 64)])
        pltpu.sync_copy(qbuf, wv_out.at[pl.ds(lo + 256, 64)])
        pltpu.sync_copy(z_acc.at[pl.ds(0, RNG * 16)],
                        z_out.at[pl.ds(lo * 16, RNG * 16)])

    return k(q, kk, v, src_s, dst_s, counts)


_BQ = 2560


def _qkv_body(h_ref, qw, qb, kw, kb, vw, vb, q_ref, k_ref, v_ref):
    hb = h_ref[...]
    q_ref[...] = jnp.dot(hb, qw[...], preferred_element_type=jnp.float32) + qb[...]
    k_ref[...] = jnp.dot(hb, kw[...], preferred_element_type=jnp.float32) + kb[...]
    v_ref[...] = jnp.dot(hb, vw[...], preferred_element_type=jnp.float32) + vb[...]


def _qkv(h, qw, qb, kw, kb, vw, vb):
    row = pl.BlockSpec((_BQ, D), lambda i: (i, 0))
    wspec = pl.BlockSpec((D, D), lambda i: (0, 0))
    bspec = pl.BlockSpec((1, D), lambda i: (0, 0))
    return pl.pallas_call(
        _qkv_body,
        grid=(NP // _BQ,),
        in_specs=[row, wspec, bspec, wspec, bspec, wspec, bspec],
        out_specs=[row, row, row],
        out_shape=[jax.ShapeDtypeStruct((NP, D), jnp.float32)] * 3,
    )(h, qw, qb.reshape(1, D), kw, kb.reshape(1, D), vw, vb.reshape(1, D))


def _ln_blk(x, s, b):
    mu = jnp.mean(x, axis=-1, keepdims=True)
    d = x - mu
    var = jnp.mean(d * d, axis=-1, keepdims=True)
    return d * jax.lax.rsqrt(var + 1e-5) * s + b


def _post_body(hin_ref, wv_ref, z_ref, ow, ob, f1w, f1b, f2w, f2b,
               l1s, l1b, l2s, l2b, out_ref):
    wv = wv_ref[...]
    z = z_ref[...]
    hsel = (lax.broadcasted_iota(jnp.int32, (16, D), 1) // DK
            == lax.broadcasted_iota(jnp.int32, (16, D), 0)).astype(jnp.float32)
    zexp = jnp.dot(z, hsel, preferred_element_type=jnp.float32)
    attn = wv / jnp.where(zexp == 0.0, 1.0, zexp)
    h1 = hin_ref[...] + jnp.dot(attn, ow[...],
                                preferred_element_type=jnp.float32) + ob[...]
    h1 = _ln_blk(h1, l1s[...], l1b[...])
    f = jnp.maximum(jnp.dot(h1, f1w[...],
                            preferred_element_type=jnp.float32) + f1b[...], 0.0)
    h2 = h1 + jnp.dot(f, f2w[...], preferred_element_type=jnp.float32) + f2b[...]
    out_ref[...] = _ln_blk(h2, l2s[...], l2b[...])


def _post(hin, wv, z, ow, ob, f1w, f1b, f2w, f2b, l1s, l1b, l2s, l2b):
    row = pl.BlockSpec((_BQ, D), lambda i: (i, 0))
    zspec = pl.BlockSpec((_BQ, 16), lambda i: (i, 0))
    bspec = pl.BlockSpec((1, D), lambda i: (0, 0))
    return pl.pallas_call(
        _post_body,
        grid=(NP // _BQ,),
        in_specs=[row, row, zspec,
                  pl.BlockSpec((D, D), lambda i: (0, 0)), bspec,
                  pl.BlockSpec((D, 2 * D), lambda i: (0, 0)),
                  pl.BlockSpec((1, 2 * D), lambda i: (0, 0)),
                  pl.BlockSpec((2 * D, D), lambda i: (0, 0)), bspec,
                  bspec, bspec, bspec, bspec],
        out_specs=row,
        out_shape=jax.ShapeDtypeStruct((NP, D), jnp.float32),
    )(hin, wv, z, ow, ob.reshape(1, D), f1w, f1b.reshape(1, 2 * D), f2w,
      f2b.reshape(1, D), l1s.reshape(1, D), l1b.reshape(1, D),
      l2s.reshape(1, D), l2b.reshape(1, D))


def kernel(x, edge_index, QW, Qb, KW, Kb, VW, Vb, OW, Ob, F1W, F1b, F2W, F2b,
           LN1s, LN1b, LN2s, LN2b):
    src = edge_index[0].astype(jnp.int32)
    dst = edge_index[1].astype(jnp.int32)
    src_s, dst_s, counts = _bucket(src, dst)
    h = jnp.pad(x, ((0, NP - N), (0, 0)))
    for l in range(2):
        q, kk, v = _qkv(h, QW[l], Qb[l], KW[l], Kb[l], VW[l], Vb[l])
        wv, z = _edge(q, kk, v, src_s, dst_s, counts)
        z = z.reshape(NP, 16)
        h = _post(h, wv, z, OW[l], Ob[l], F1W[l], F1b[l], F2W[l], F2b[l],
                  LN1s[l], LN1b[l], LN2s[l], LN2b[l])
    return h[:N]


# speedup vs baseline: 4.5350x; 4.5350x over previous
import dataclasses
import functools

import jax
import jax.numpy as jnp
from jax import lax
from jax.experimental import pallas as pl
from jax.experimental.pallas import tpu as pltpu
from jax.experimental.pallas import tpu_sc as plsc

N = 10000
E = 320000
D = 128
H = 8
DK = 16
NT = 32
RNG = 320
NP = NT * RNG
CAP = 11264
ECHUNK = 1280
CHUNK = 128

_mesh = plsc.VectorSubcoreMesh(core_axis_name="c", subcore_axis_name="s")

_sc_params = pltpu.CompilerParams()
if "needs_layout_passes" in pltpu.CompilerParams.__dataclass_fields__:
    _sc_params = dataclasses.replace(_sc_params, needs_layout_passes=False)


def _wid():
    return lax.axis_index("s") * 2 + lax.axis_index("c")


def _bucket(src, dst):
    @functools.partial(
        pl.kernel,
        out_type=(
            jax.ShapeDtypeStruct((NT, CAP), jnp.int32),
            jax.ShapeDtypeStruct((NT, CAP), jnp.int32),
            jax.ShapeDtypeStruct((NT, 128), jnp.int32),
        ),
        mesh=_mesh,
        scratch_types=[
            pltpu.VMEM((ECHUNK,), jnp.int32),
            pltpu.VMEM((ECHUNK,), jnp.int32),
            pltpu.VMEM((CAP,), jnp.int32),
            pltpu.VMEM((CAP,), jnp.int32),
            pltpu.VMEM((128,), jnp.int32),
            pltpu.SemaphoreType.DMA,
        ],
        compiler_params=_sc_params,
    )
    def k(src_hbm, dst_hbm, src_out, dst_out, cnt_out, sbuf, dbuf, ssel, dsel,
          cntv, sem):
        t = _wid()
        lo = t * RNG
        hi = lo + RNG
        zero = jnp.zeros((16,), jnp.int32)

        hivec = jnp.full((16,), hi, jnp.int32)

        @pl.loop(0, CAP // 16)
        def _(i):
            ssel[pl.ds(i * 16, 16)] = zero
            dsel[pl.ds(i * 16, 16)] = hivec

        def chunk_body(c, off):
            pltpu.sync_copy(src_hbm.at[pl.ds(c * ECHUNK, ECHUNK)], sbuf)
            pltpu.sync_copy(dst_hbm.at[pl.ds(c * ECHUNK, ECHUNK)], dbuf)

            def vec_body(j, off):
                dvec = dbuf[pl.ds(j * 16, 16)]
                svec = sbuf[pl.ds(j * 16, 16)]
                m = (dvec >= lo) & (dvec < hi)
                off = jnp.minimum(off, CAP - 16)
                plsc.store_compressed(ssel.at[pl.ds(off, 16)], svec, mask=m)
                plsc.store_compressed(dsel.at[pl.ds(off, 16)], dvec, mask=m)
                return off + jnp.sum(m.astype(jnp.int32))

            return lax.fori_loop(0, ECHUNK // 16, vec_body, off)

        cnt = lax.fori_loop(0, E // ECHUNK, chunk_body, jnp.int32(0))
        pltpu.sync_copy(ssel, src_out.at[t])
        pltpu.sync_copy(dsel, dst_out.at[t])
        cv = jnp.full((16,), cnt, jnp.int32)

        @pl.loop(0, 8)
        def _(i):
            cntv[pl.ds(i * 16, 16)] = cv

        pltpu.sync_copy(cntv, cnt_out.at[t])

    return k(src, dst)


def _edge(q, kk, v, src_s, dst_s, counts):
    @functools.partial(
        pl.kernel,
        out_type=(
            jax.ShapeDtypeStruct((NP, D), jnp.float32),
            jax.ShapeDtypeStruct((NP * 16,), jnp.float32),
        ),
        mesh=_mesh,
        scratch_types=(
            [pltpu.VMEM(((RNG + 8) * DK,), jnp.float32) for _ in range(H)] + [
                pltpu.VMEM(((RNG + 8) * 16,), jnp.float32),
                pltpu.VMEM((CHUNK,), jnp.int32),
                pltpu.VMEM((CHUNK,), jnp.int32),
                pltpu.VMEM((CHUNK // 2, D), jnp.float32),
                pltpu.VMEM((CHUNK // 2, D), jnp.float32),
                pltpu.VMEM((CHUNK // 2, D), jnp.float32),
                pltpu.SemaphoreType.DMA,
            ]
        ),
        compiler_params=_sc_params,
    )
    def k(q_hbm, k_hbm, v_hbm, src_hbm, dst_hbm, cnt_hbm, wv_out, z_out,
          a0, a1, a2, a3, a4, a5, a6, a7, z_acc, sidx, didx, kbuf, qbuf,
          vbuf, sem):
        acc = (a0, a1, a2, a3, a4, a5, a6, a7)
        t = _wid()
        lo = t * RNG
        zf = jnp.zeros((16,), jnp.float32)
        lane = lax.iota(jnp.int32, 16)
        x8 = lane ^ 8
        x4 = lane ^ 4
        x2 = lane ^ 2
        x1 = lane ^ 1
        m8 = (lane & 8) == 0
        m4 = (lane & 4) == 0
        m2 = (lane & 2) == 0
        bmap = [jnp.full((16,), ((hh & 1) << 3) | (((hh >> 1) & 1) << 2)
                | (((hh >> 2) & 1) << 1), jnp.int32) for hh in range(H)]
        zmap = (((lane & 1) << 3) + (((lane >> 1) & 1) << 2)
                + (((lane >> 2) & 1) << 1))

        def _sh(p, x):
            return p + p.at[x].get(mode="promise_in_bounds")

        @pl.loop(0, RNG + 8)
        def _(r):
            for hh in range(H):
                acc[hh][pl.ds(r * 16, 16)] = zf
            z_acc[pl.ds(r * 16, 16)] = zf

        pltpu.sync_copy(cnt_hbm.at[t], didx)
        n = didx[pl.ds(0, 16)][0]
        nch = (n + (CHUNK - 1)) // CHUNK

        @pl.loop(0, nch)
        def _(c):
            base = c * CHUNK
            pltpu.sync_copy(src_hbm.at[t, pl.ds(base, CHUNK)], sidx)
            pltpu.sync_copy(dst_hbm.at[t, pl.ds(base, CHUNK)], didx)
            for half in range(2):
                hb = half * (CHUNK // 2)
                si = sidx.at[pl.ds(hb, CHUNK // 2)]
                di = didx.at[pl.ds(hb, CHUNK // 2)]
                cp1 = pltpu.async_copy(k_hbm.at[si], kbuf, sem)
                cp2 = pltpu.async_copy(q_hbm.at[di], qbuf, sem)
                cp3 = pltpu.async_copy(v_hbm.at[si], vbuf, sem)
                cp1.wait()
                cp2.wait()
                cp3.wait()

                @pl.loop(0, CHUNK // 32)
                def _(g):
                    dlvec = didx[pl.ds(hb + g * 16, 16)] - lo
                    for jj in range(0, 16, 2):
                        rows = (g * 16 + jj, g * 16 + jj + 1)
                        dls = (dlvec[jj], dlvec[jj + 1])
                        kq = [kbuf[r, pl.ds(hh * DK, DK)]
                              * qbuf[r, pl.ds(hh * DK, DK)]
                              for r in rows for hh in range(H)]
                        a = [_sh(p, x8) for p in kq]
                        b = [jnp.where(m8, a[e * 8 + 2 * j], a[e * 8 + 2 * j + 1])
                             for e in range(2) for j in range(4)]
                        cc = [_sh(y, x4) for y in b]
                        dd = [jnp.where(m4, cc[e * 4 + 2 * j],
                                        cc[e * 4 + 2 * j + 1])
                              for e in range(2) for j in range(2)]
                        ee = [_sh(y, x2) for y in dd]
                        ff = [jnp.where(m2, ee[e * 2], ee[e * 2 + 1])
                              for e in range(2)]
                        gg = [_sh(y, x1) for y in ff]
                        sc = [jnp.exp(jnp.clip(y * 0.25, -5.0, 5.0))
                              for y in gg]
                        wvs = [vbuf[rows[e], pl.ds(hh * DK, DK)]
                               * sc[e].at[bmap[hh]].get(
                                   mode="promise_in_bounds")
                               for e in range(2) for hh in range(H)]
                        zrows = [jnp.where(lane < 8,
                                           sc[e].at[zmap].get(
                                               mode="promise_in_bounds"),
                                           0.0)
                                 for e in range(2)]
                        for e in range(2):
                            zslc = pl.ds(dls[e] * 16, 16)
                            for hh in range(H):
                                plsc.addupdate(acc[hh].at[zslc],
                                               wvs[e * 8 + hh])
                            plsc.addupdate(z_acc.at[zslc], zrows[e])

        @pl.loop(0, 64)
        def _(r):
            for buf, off in ((kbuf, 0), (qbuf, 64), (vbuf, 128)):
                for hh in range(H):
                    buf[r, pl.ds(hh * DK, DK)] = acc[hh][
                        pl.ds((off + r) * 16, 16)]

        pltpu.sync_copy(kbuf, wv_out.at[pl.ds(lo, 64)])
        pltpu.sync_copy(qbuf, wv_out.at[pl.ds(lo + 64, 64)])
        pltpu.sync_copy(vbuf, wv_out.at[pl.ds(lo + 128, 64)])

        @pl.loop(0, 64)
        def _(r):
            for buf, off in ((kbuf, 192), (qbuf, 256)):
                for hh in range(H):
                    buf[r, pl.ds(hh * DK, DK)] = acc[hh][
                        pl.ds((off + r) * 16, 16)]

        pltpu.sync_copy(kbuf, wv_out.at[pl.ds(lo + 192, 64)])
        pltpu.sync_copy(qbuf, wv_out.at[pl.ds(lo + 256, 64)])
        pltpu.sync_copy(z_acc.at[pl.ds(0, RNG * 16)],
                        z_out.at[pl.ds(lo * 16, RNG * 16)])

    return k(q, kk, v, src_s, dst_s, counts)


_BQ = 2560


def _qkv_body(h_ref, qw, qb, kw, kb, vw, vb, q_ref, k_ref, v_ref):
    hb = h_ref[...]
    q_ref[...] = jnp.dot(hb, qw[...], preferred_element_type=jnp.float32) + qb[...]
    k_ref[...] = jnp.dot(hb, kw[...], preferred_element_type=jnp.float32) + kb[...]
    v_ref[...] = jnp.dot(hb, vw[...], preferred_element_type=jnp.float32) + vb[...]


def _qkv(h, qw, qb, kw, kb, vw, vb):
    row = pl.BlockSpec((_BQ, D), lambda i: (i, 0))
    wspec = pl.BlockSpec((D, D), lambda i: (0, 0))
    bspec = pl.BlockSpec((1, D), lambda i: (0, 0))
    return pl.pallas_call(
        _qkv_body,
        grid=(NP // _BQ,),
        in_specs=[row, wspec, bspec, wspec, bspec, wspec, bspec],
        out_specs=[row, row, row],
        out_shape=[jax.ShapeDtypeStruct((NP, D), jnp.float32)] * 3,
    )(h, qw, qb.reshape(1, D), kw, kb.reshape(1, D), vw, vb.reshape(1, D))


def _ln_blk(x, s, b):
    mu = jnp.mean(x, axis=-1, keepdims=True)
    d = x - mu
    var = jnp.mean(d * d, axis=-1, keepdims=True)
    return d * jax.lax.rsqrt(var + 1e-5) * s + b


def _post_body(hin_ref, wv_ref, z_ref, ow, ob, f1w, f1b, f2w, f2b,
               l1s, l1b, l2s, l2b, out_ref):
    wv = wv_ref[...]
    z = z_ref[...]
    hsel = (lax.broadcasted_iota(jnp.int32, (16, D), 1) // DK
            == lax.broadcasted_iota(jnp.int32, (16, D), 0)).astype(jnp.float32)
    zexp = jnp.dot(z, hsel, preferred_element_type=jnp.float32)
    attn = wv / jnp.where(zexp == 0.0, 1.0, zexp)
    h1 = hin_ref[...] + jnp.dot(attn, ow[...],
                                preferred_element_type=jnp.float32) + ob[...]
    h1 = _ln_blk(h1, l1s[...], l1b[...])
    f = jnp.maximum(jnp.dot(h1, f1w[...],
                            preferred_element_type=jnp.float32) + f1b[...], 0.0)
    h2 = h1 + jnp.dot(f, f2w[...], preferred_element_type=jnp.float32) + f2b[...]
    out_ref[...] = _ln_blk(h2, l2s[...], l2b[...])


def _post(hin, wv, z, ow, ob, f1w, f1b, f2w, f2b, l1s, l1b, l2s, l2b):
    row = pl.BlockSpec((_BQ, D), lambda i: (i, 0))
    zspec = pl.BlockSpec((_BQ, 16), lambda i: (i, 0))
    bspec = pl.BlockSpec((1, D), lambda i: (0, 0))
    return pl.pallas_call(
        _post_body,
        grid=(NP // _BQ,),
        in_specs=[row, row, zspec,
                  pl.BlockSpec((D, D), lambda i: (0, 0)), bspec,
                  pl.BlockSpec((D, 2 * D), lambda i: (0, 0)),
                  pl.BlockSpec((1, 2 * D), lambda i: (0, 0)),
                  pl.BlockSpec((2 * D, D), lambda i: (0, 0)), bspec,
                  bspec, bspec, bspec, bspec],
        out_specs=row,
        out_shape=jax.ShapeDtypeStruct((NP, D), jnp.float32),
    )(hin, wv, z, ow, ob.reshape(1, D), f1w, f1b.reshape(1, 2 * D), f2w,
      f2b.reshape(1, D), l1s.reshape(1, D), l1b.reshape(1, D),
      l2s.reshape(1, D), l2b.reshape(1, D))


def kernel(x, edge_index, QW, Qb, KW, Kb, VW, Vb, OW, Ob, F1W, F1b, F2W, F2b,
           LN1s, LN1b, LN2s, LN2b):
    src = edge_index[0].astype(jnp.int32)
    dst = edge_index[1].astype(jnp.int32)
    src_s, dst_s, counts = _bucket(src, dst)
    h = jnp.pad(x, ((0, NP - N), (0, 0)))
    for l in range(2):
        q, kk, v = _qkv(h, QW[l], Qb[l], KW[l], Kb[l], VW[l], Vb[l])
        wv, z = _edge(q, kk, v, src_s, dst_s, counts)
        z = z.reshape(NP, 16)
        h = _post(h, wv, z, OW[l], Ob[l], F1W[l], F1b[l], F2W[l], F2b[l],
                  LN1s[l], LN1b[l], LN2s[l], LN2b[l])
    return h[:N]


# speedup vs baseline: 5.4521x; 1.2022x over previous
import dataclasses
import functools

import jax
import jax.numpy as jnp
from jax import lax
from jax.experimental import pallas as pl
from jax.experimental.pallas import tpu as pltpu
from jax.experimental.pallas import tpu_sc as plsc

N = 10000
E = 320000
D = 128
H = 8
DK = 16
NT = 32
RNG = 320
NP = NT * RNG
CAP = 11264
ECHUNK = 6400
CHUNK = 128

_mesh = plsc.VectorSubcoreMesh(core_axis_name="c", subcore_axis_name="s")

_sc_params = pltpu.CompilerParams()
if "needs_layout_passes" in pltpu.CompilerParams.__dataclass_fields__:
    _sc_params = dataclasses.replace(_sc_params, needs_layout_passes=False)


def _wid():
    return lax.axis_index("s") * 2 + lax.axis_index("c")


def _bucket(src, dst):
    @functools.partial(
        pl.kernel,
        out_type=(
            jax.ShapeDtypeStruct((NT, CAP), jnp.int32),
            jax.ShapeDtypeStruct((NT, CAP), jnp.int32),
            jax.ShapeDtypeStruct((NT, 128), jnp.int32),
        ),
        mesh=_mesh,
        scratch_types=[
            pltpu.VMEM((ECHUNK,), jnp.int32),
            pltpu.VMEM((ECHUNK,), jnp.int32),
            pltpu.VMEM((CAP,), jnp.int32),
            pltpu.VMEM((CAP,), jnp.int32),
            pltpu.VMEM((128,), jnp.int32),
            pltpu.SemaphoreType.DMA,
        ],
        compiler_params=_sc_params,
    )
    def k(src_hbm, dst_hbm, src_out, dst_out, cnt_out, sbuf, dbuf, ssel, dsel,
          cntv, sem):
        t = _wid()
        lo = t * RNG
        hi = lo + RNG
        zero = jnp.zeros((16,), jnp.int32)

        hivec = jnp.full((16,), hi, jnp.int32)

        @pl.loop(0, CAP // 16)
        def _(i):
            ssel[pl.ds(i * 16, 16)] = zero
            dsel[pl.ds(i * 16, 16)] = hivec

        def chunk_body(c, off):
            pltpu.sync_copy(src_hbm.at[pl.ds(c * ECHUNK, ECHUNK)], sbuf)
            pltpu.sync_copy(dst_hbm.at[pl.ds(c * ECHUNK, ECHUNK)], dbuf)

            def vec_body(j, off):
                dv = [dbuf[pl.ds((j * 4 + i) * 16, 16)] for i in range(4)]
                sv = [sbuf[pl.ds((j * 4 + i) * 16, 16)] for i in range(4)]
                ms = [(d >= lo) & (d < hi) for d in dv]
                cs = [plsc.all_reduce_population_count(m)[0] for m in ms]
                off = jnp.minimum(off, CAP - 64)
                o = off
                for i in range(4):
                    plsc.store_compressed(ssel.at[pl.ds(o, 16)], sv[i],
                                          mask=ms[i])
                    plsc.store_compressed(dsel.at[pl.ds(o, 16)], dv[i],
                                          mask=ms[i])
                    o = o + cs[i]
                return o

            return lax.fori_loop(0, ECHUNK // 64, vec_body, off)

        cnt = lax.fori_loop(0, E // ECHUNK, chunk_body, jnp.int32(0))
        pltpu.sync_copy(ssel, src_out.at[t])
        pltpu.sync_copy(dsel, dst_out.at[t])
        cv = jnp.full((16,), cnt, jnp.int32)

        @pl.loop(0, 8)
        def _(i):
            cntv[pl.ds(i * 16, 16)] = cv

        pltpu.sync_copy(cntv, cnt_out.at[t])

    return k(src, dst)


def _edge(q, kk, v, src_s, dst_s, counts):
    @functools.partial(
        pl.kernel,
        out_type=(
            jax.ShapeDtypeStruct((NP, D), jnp.float32),
            jax.ShapeDtypeStruct((NP * 16,), jnp.float32),
        ),
        mesh=_mesh,
        scratch_types=(
            [pltpu.VMEM(((RNG + 8) * DK,), jnp.float32) for _ in range(H)] + [
                pltpu.VMEM(((RNG + 8) * 16,), jnp.float32),
                pltpu.VMEM((CHUNK,), jnp.int32),
                pltpu.VMEM((CHUNK,), jnp.int32),
                pltpu.VMEM((CHUNK // 2, D), jnp.float32),
                pltpu.VMEM((CHUNK // 2, D), jnp.float32),
                pltpu.VMEM((CHUNK // 2, D), jnp.float32),
                pltpu.SemaphoreType.DMA,
            ]
        ),
        compiler_params=_sc_params,
    )
    def k(q_hbm, k_hbm, v_hbm, src_hbm, dst_hbm, cnt_hbm, wv_out, z_out,
          a0, a1, a2, a3, a4, a5, a6, a7, z_acc, sidx, didx, kbuf, qbuf,
          vbuf, sem):
        acc = (a0, a1, a2, a3, a4, a5, a6, a7)
        t = _wid()
        lo = t * RNG
        zf = jnp.zeros((16,), jnp.float32)
        lane = lax.iota(jnp.int32, 16)
        x8 = lane ^ 8
        x4 = lane ^ 4
        x2 = lane ^ 2
        x1 = lane ^ 1
        m8 = (lane & 8) == 0
        m4 = (lane & 4) == 0
        m2 = (lane & 2) == 0
        bmap = [jnp.full((16,), ((hh & 1) << 3) | (((hh >> 1) & 1) << 2)
                | (((hh >> 2) & 1) << 1), jnp.int32) for hh in range(H)]
        zmap = (((lane & 1) << 3) + (((lane >> 1) & 1) << 2)
                + (((lane >> 2) & 1) << 1))

        def _sh(p, x):
            return p + p.at[x].get(mode="promise_in_bounds")

        @pl.loop(0, RNG + 8)
        def _(r):
            for hh in range(H):
                acc[hh][pl.ds(r * 16, 16)] = zf
            z_acc[pl.ds(r * 16, 16)] = zf

        pltpu.sync_copy(cnt_hbm.at[t], didx)
        n = didx[pl.ds(0, 16)][0]
        nch = (n + (CHUNK - 1)) // CHUNK

        @pl.loop(0, nch)
        def _(c):
            base = c * CHUNK
            pltpu.sync_copy(src_hbm.at[t, pl.ds(base, CHUNK)], sidx)
            pltpu.sync_copy(dst_hbm.at[t, pl.ds(base, CHUNK)], didx)
            for half in range(2):
                hb = half * (CHUNK // 2)
                si = sidx.at[pl.ds(hb, CHUNK // 2)]
                di = didx.at[pl.ds(hb, CHUNK // 2)]
                cp1 = pltpu.async_copy(k_hbm.at[si], kbuf, sem)
                cp2 = pltpu.async_copy(q_hbm.at[di], qbuf, sem)
                cp3 = pltpu.async_copy(v_hbm.at[si], vbuf, sem)
                cp1.wait()
                cp2.wait()
                cp3.wait()

                @pl.loop(0, CHUNK // 32)
                def _(g):
                    dlvec = didx[pl.ds(hb + g * 16, 16)] - lo
                    for jj in range(0, 16, 2):
                        rows = (g * 16 + jj, g * 16 + jj + 1)
                        dls = (dlvec[jj], dlvec[jj + 1])
                        kq = [kbuf[r, pl.ds(hh * DK, DK)]
                              * qbuf[r, pl.ds(hh * DK, DK)]
                              for r in rows for hh in range(H)]
                        a = [_sh(p, x8) for p in kq]
                        b = [jnp.where(m8, a[e * 8 + 2 * j], a[e * 8 + 2 * j + 1])
                             for e in range(2) for j in range(4)]
                        cc = [_sh(y, x4) for y in b]
                        dd = [jnp.where(m4, cc[e * 4 + 2 * j],
                                        cc[e * 4 + 2 * j + 1])
                              for e in range(2) for j in range(2)]
                        ee = [_sh(y, x2) for y in dd]
                        ff = [jnp.where(m2, ee[e * 2], ee[e * 2 + 1])
                              for e in range(2)]
                        gg = [_sh(y, x1) for y in ff]
                        sc = [jnp.exp(jnp.clip(y * 0.25, -5.0, 5.0))
                              for y in gg]
                        wvs = [vbuf[rows[e], pl.ds(hh * DK, DK)]
                               * sc[e].at[bmap[hh]].get(
                                   mode="promise_in_bounds")
                               for e in range(2) for hh in range(H)]
                        zrows = [jnp.where(lane < 8,
                                           sc[e].at[zmap].get(
                                               mode="promise_in_bounds"),
                                           0.0)
                                 for e in range(2)]
                        for e in range(2):
                            zslc = pl.ds(dls[e] * 16, 16)
                            for hh in range(H):
                                plsc.addupdate(acc[hh].at[zslc],
                                               wvs[e * 8 + hh])
                            plsc.addupdate(z_acc.at[zslc], zrows[e])

        @pl.loop(0, 64)
        def _(r):
            for buf, off in ((kbuf, 0), (qbuf, 64), (vbuf, 128)):
                for hh in range(H):
                    buf[r, pl.ds(hh * DK, DK)] = acc[hh][
                        pl.ds((off + r) * 16, 16)]

        pltpu.sync_copy(kbuf, wv_out.at[pl.ds(lo, 64)])
        pltpu.sync_copy(qbuf, wv_out.at[pl.ds(lo + 64, 64)])
        pltpu.sync_copy(vbuf, wv_out.at[pl.ds(lo + 128, 64)])

        @pl.loop(0, 64)
        def _(r):
            for buf, off in ((kbuf, 192), (qbuf, 256)):
                for hh in range(H):
                    buf[r, pl.ds(hh * DK, DK)] = acc[hh][
                        pl.ds((off + r) * 16, 16)]

        pltpu.sync_copy(kbuf, wv_out.at[pl.ds(lo + 192, 64)])
        pltpu.sync_copy(qbuf, wv_out.at[pl.ds(lo + 256, 64)])
        pltpu.sync_copy(z_acc.at[pl.ds(0, RNG * 16)],
                        z_out.at[pl.ds(lo * 16, RNG * 16)])

    return k(q, kk, v, src_s, dst_s, counts)


_BQ = 2560


def _qkv_body(h_ref, qw, qb, kw, kb, vw, vb, q_ref, k_ref, v_ref):
    hb = h_ref[...]
    q_ref[...] = jnp.dot(hb, qw[...], preferred_element_type=jnp.float32) + qb[...]
    k_ref[...] = jnp.dot(hb, kw[...], preferred_element_type=jnp.float32) + kb[...]
    v_ref[...] = jnp.dot(hb, vw[...], preferred_element_type=jnp.float32) + vb[...]


def _qkv(h, qw, qb, kw, kb, vw, vb):
    row = pl.BlockSpec((_BQ, D), lambda i: (i, 0))
    wspec = pl.BlockSpec((D, D), lambda i: (0, 0))
    bspec = pl.BlockSpec((1, D), lambda i: (0, 0))
    return pl.pallas_call(
        _qkv_body,
        grid=(NP // _BQ,),
        in_specs=[row, wspec, bspec, wspec, bspec, wspec, bspec],
        out_specs=[row, row, row],
        out_shape=[jax.ShapeDtypeStruct((NP, D), jnp.float32)] * 3,
    )(h, qw, qb.reshape(1, D), kw, kb.reshape(1, D), vw, vb.reshape(1, D))


def _ln_blk(x, s, b):
    mu = jnp.mean(x, axis=-1, keepdims=True)
    d = x - mu
    var = jnp.mean(d * d, axis=-1, keepdims=True)
    return d * jax.lax.rsqrt(var + 1e-5) * s + b


def _post_body(hin_ref, wv_ref, z_ref, ow, ob, f1w, f1b, f2w, f2b,
               l1s, l1b, l2s, l2b, out_ref):
    wv = wv_ref[...]
    z = z_ref[...]
    hsel = (lax.broadcasted_iota(jnp.int32, (16, D), 1) // DK
            == lax.broadcasted_iota(jnp.int32, (16, D), 0)).astype(jnp.float32)
    zexp = jnp.dot(z, hsel, preferred_element_type=jnp.float32)
    attn = wv / jnp.where(zexp == 0.0, 1.0, zexp)
    h1 = hin_ref[...] + jnp.dot(attn, ow[...],
                                preferred_element_type=jnp.float32) + ob[...]
    h1 = _ln_blk(h1, l1s[...], l1b[...])
    f = jnp.maximum(jnp.dot(h1, f1w[...],
                            preferred_element_type=jnp.float32) + f1b[...], 0.0)
    h2 = h1 + jnp.dot(f, f2w[...], preferred_element_type=jnp.float32) + f2b[...]
    out_ref[...] = _ln_blk(h2, l2s[...], l2b[...])


def _post(hin, wv, z, ow, ob, f1w, f1b, f2w, f2b, l1s, l1b, l2s, l2b):
    row = pl.BlockSpec((_BQ, D), lambda i: (i, 0))
    zspec = pl.BlockSpec((_BQ, 16), lambda i: (i, 0))
    bspec = pl.BlockSpec((1, D), lambda i: (0, 0))
    return pl.pallas_call(
        _post_body,
        grid=(NP // _BQ,),
        in_specs=[row, row, zspec,
                  pl.BlockSpec((D, D), lambda i: (0, 0)), bspec,
                  pl.BlockSpec((D, 2 * D), lambda i: (0, 0)),
                  pl.BlockSpec((1, 2 * D), lambda i: (0, 0)),
                  pl.BlockSpec((2 * D, D), lambda i: (0, 0)), bspec,
                  bspec, bspec, bspec, bspec],
        out_specs=row,
        out_shape=jax.ShapeDtypeStruct((NP, D), jnp.float32),
    )(hin, wv, z, ow, ob.reshape(1, D), f1w, f1b.reshape(1, 2 * D), f2w,
      f2b.reshape(1, D), l1s.reshape(1, D), l1b.reshape(1, D),
      l2s.reshape(1, D), l2b.reshape(1, D))


def kernel(x, edge_index, QW, Qb, KW, Kb, VW, Vb, OW, Ob, F1W, F1b, F2W, F2b,
           LN1s, LN1b, LN2s, LN2b):
    src = edge_index[0].astype(jnp.int32)
    dst = edge_index[1].astype(jnp.int32)
    src_s, dst_s, counts = _bucket(src, dst)
    h = jnp.pad(x, ((0, NP - N), (0, 0)))
    for l in range(2):
        q, kk, v = _qkv(h, QW[l], Qb[l], KW[l], Kb[l], VW[l], Vb[l])
        wv, z = _edge(q, kk, v, src_s, dst_s, counts)
        z = z.reshape(NP, 16)
        h = _post(h, wv, z, OW[l], Ob[l], F1W[l], F1b[l], F2W[l], F2b[l],
                  LN1s[l], LN1b[l], LN2s[l], LN2b[l])
    return h[:N]


# speedup vs baseline: 6.0253x; 1.1051x over previous
import dataclasses
import functools

import jax
import jax.numpy as jnp
from jax import lax
from jax.experimental import pallas as pl
from jax.experimental.pallas import tpu as pltpu
from jax.experimental.pallas import tpu_sc as plsc

N = 10000
E = 320000
D = 128
H = 8
DK = 16
NT = 32
RNG = 320
NP = NT * RNG
CAP = 11264
ECHUNK = 6400
CHUNK = 128

_mesh = plsc.VectorSubcoreMesh(core_axis_name="c", subcore_axis_name="s")

_sc_params = pltpu.CompilerParams()
if "needs_layout_passes" in pltpu.CompilerParams.__dataclass_fields__:
    _sc_params = dataclasses.replace(_sc_params, needs_layout_passes=False)


def _wid():
    return lax.axis_index("s") * 2 + lax.axis_index("c")


def _bucket(src, dst):
    @functools.partial(
        pl.kernel,
        out_type=(
            jax.ShapeDtypeStruct((NT, CAP), jnp.int32),
            jax.ShapeDtypeStruct((NT, 128), jnp.int32),
        ),
        mesh=_mesh,
        scratch_types=[
            pltpu.VMEM((ECHUNK,), jnp.int32),
            pltpu.VMEM((ECHUNK,), jnp.int32),
            pltpu.VMEM((CAP,), jnp.int32),
            pltpu.VMEM((128,), jnp.int32),
            pltpu.SemaphoreType.DMA,
        ],
        compiler_params=_sc_params,
    )
    def k(src_hbm, dst_hbm, pck_out, cnt_out, sbuf, dbuf, psel,
          cntv, sem):
        t = _wid()
        lo = t * RNG
        hi = lo + RNG

        hivec = jnp.full((16,), hi << 16, jnp.int32)

        @pl.loop(0, CAP // 16)
        def _(i):
            psel[pl.ds(i * 16, 16)] = hivec

        def chunk_body(c, off):
            pltpu.sync_copy(src_hbm.at[pl.ds(c * ECHUNK, ECHUNK)], sbuf)
            pltpu.sync_copy(dst_hbm.at[pl.ds(c * ECHUNK, ECHUNK)], dbuf)

            def vec_body(j, off):
                dv = [dbuf[pl.ds((j * 4 + i) * 16, 16)] for i in range(4)]
                sv = [sbuf[pl.ds((j * 4 + i) * 16, 16)] for i in range(4)]
                ms = [(d >= lo) & (d < hi) for d in dv]
                pv = [sv[i] | (dv[i] << 16) for i in range(4)]
                cs = [plsc.all_reduce_population_count(m)[0] for m in ms]
                off = jnp.minimum(off, CAP - 64)
                o = off
                for i in range(4):
                    plsc.store_compressed(psel.at[pl.ds(o, 16)], pv[i],
                                          mask=ms[i])
                    o = o + cs[i]
                return o

            return lax.fori_loop(0, ECHUNK // 64, vec_body, off)

        cnt = lax.fori_loop(0, E // ECHUNK, chunk_body, jnp.int32(0))
        pltpu.sync_copy(psel, pck_out.at[t])
        cv = jnp.full((16,), cnt, jnp.int32)

        @pl.loop(0, 8)
        def _(i):
            cntv[pl.ds(i * 16, 16)] = cv

        pltpu.sync_copy(cntv, cnt_out.at[t])

    return k(src, dst)


def _edge(q, kv, pck, counts):
    @functools.partial(
        pl.kernel,
        out_type=(
            jax.ShapeDtypeStruct((NP, D), jnp.float32),
            jax.ShapeDtypeStruct((NP * 16,), jnp.float32),
        ),
        mesh=_mesh,
        scratch_types=(
            [pltpu.VMEM(((RNG + 8) * DK,), jnp.float32) for _ in range(H)] + [
                pltpu.VMEM(((RNG + 8) * 16,), jnp.float32),
                pltpu.VMEM((CHUNK,), jnp.int32),
                pltpu.VMEM((CHUNK,), jnp.int32),
                pltpu.VMEM((CHUNK,), jnp.int32),
                pltpu.VMEM((CHUNK // 2, 2 * D), jnp.float32),
                pltpu.VMEM((CHUNK // 2, 2 * D), jnp.float32),
                pltpu.VMEM((CHUNK // 2, D), jnp.float32),
                pltpu.VMEM((CHUNK // 2, D), jnp.float32),
                pltpu.SemaphoreType.DMA,
                pltpu.SemaphoreType.DMA,
            ]
        ),
        compiler_params=_sc_params,
    )
    def k(q_hbm, kv_hbm, pck_hbm, cnt_hbm, wv_out, z_out,
          a0, a1, a2, a3, a4, a5, a6, a7, z_acc, pbuf, sidx, didx,
          kvA, kvB, qA, qB, semA, semB):
        acc = (a0, a1, a2, a3, a4, a5, a6, a7)
        t = _wid()
        lo = t * RNG
        zf = jnp.zeros((16,), jnp.float32)
        lane = lax.iota(jnp.int32, 16)
        x8 = lane ^ 8
        x4 = lane ^ 4
        x2 = lane ^ 2
        x1 = lane ^ 1
        m8 = (lane & 8) == 0
        m4 = (lane & 4) == 0
        m2 = (lane & 2) == 0
        bmap = [jnp.full((16,), ((hh & 1) << 3) | (((hh >> 1) & 1) << 2)
                | (((hh >> 2) & 1) << 1), jnp.int32) for hh in range(H)]
        zmap = (((lane & 1) << 3) + (((lane >> 1) & 1) << 2)
                + (((lane >> 2) & 1) << 1))

        def _sh(p, x):
            return p + p.at[x].get(mode="promise_in_bounds")

        @pl.loop(0, RNG + 8)
        def _(r):
            for hh in range(H):
                acc[hh][pl.ds(r * 16, 16)] = zf
            z_acc[pl.ds(r * 16, 16)] = zf

        pltpu.sync_copy(cnt_hbm.at[t], pbuf)
        n = pbuf[pl.ds(0, 16)][0]
        nch = (n + (CHUNK - 1)) // CHUNK

        @pl.loop(0, nch)
        def _(c):
            base = c * CHUNK
            pltpu.sync_copy(pck_hbm.at[t, pl.ds(base, CHUNK)], pbuf)

            @pl.loop(0, CHUNK // 16)
            def _(i):
                pv = pbuf[pl.ds(i * 16, 16)]
                sidx[pl.ds(i * 16, 16)] = pv & 0xFFFF
                didx[pl.ds(i * 16, 16)] = lax.shift_right_logical(pv, 16)

            cpA = (pltpu.async_copy(kv_hbm.at[sidx.at[pl.ds(0, CHUNK // 2)]],
                                    kvA, semA),
                   pltpu.async_copy(q_hbm.at[didx.at[pl.ds(0, CHUNK // 2)]],
                                    qA, semA))
            cpB = (pltpu.async_copy(
                       kv_hbm.at[sidx.at[pl.ds(CHUNK // 2, CHUNK // 2)]],
                       kvB, semB),
                   pltpu.async_copy(
                       q_hbm.at[didx.at[pl.ds(CHUNK // 2, CHUNK // 2)]],
                       qB, semB))
            for half in range(2):
                hb = half * (CHUNK // 2)
                kvbuf, qbuf = (kvA, qA) if half == 0 else (kvB, qB)
                for cp in (cpA if half == 0 else cpB):
                    cp.wait()

                @pl.loop(0, CHUNK // 32)
                def _(g):
                    dlvec = didx[pl.ds(hb + g * 16, 16)] - lo
                    for jj in range(0, 16, 2):
                        rows = (g * 16 + jj, g * 16 + jj + 1)
                        dls = (dlvec[jj], dlvec[jj + 1])
                        kq = [kvbuf[r, pl.ds(hh * DK, DK)]
                              * qbuf[r, pl.ds(hh * DK, DK)]
                              for r in rows for hh in range(H)]
                        a = [_sh(p, x8) for p in kq]
                        b = [jnp.where(m8, a[e * 8 + 2 * j], a[e * 8 + 2 * j + 1])
                             for e in range(2) for j in range(4)]
                        cc = [_sh(y, x4) for y in b]
                        dd = [jnp.where(m4, cc[e * 4 + 2 * j],
                                        cc[e * 4 + 2 * j + 1])
                              for e in range(2) for j in range(2)]
                        ee = [_sh(y, x2) for y in dd]
                        ff = [jnp.where(m2, ee[e * 2], ee[e * 2 + 1])
                              for e in range(2)]
                        gg = [_sh(y, x1) for y in ff]
                        sc = [jnp.exp(jnp.clip(y * 0.25, -5.0, 5.0))
                              for y in gg]
                        wvs = [kvbuf[rows[e], pl.ds(D + hh * DK, DK)]
                               * sc[e].at[bmap[hh]].get(
                                   mode="promise_in_bounds")
                               for e in range(2) for hh in range(H)]
                        zrows = [jnp.where(lane < 8,
                                           sc[e].at[zmap].get(
                                               mode="promise_in_bounds"),
                                           0.0)
                                 for e in range(2)]
                        for e in range(2):
                            zslc = pl.ds(dls[e] * 16, 16)
                            for hh in range(H):
                                plsc.addupdate(acc[hh].at[zslc],
                                               wvs[e * 8 + hh])
                            plsc.addupdate(z_acc.at[zslc], zrows[e])

        for m in range(5):
            buf = qA if m % 2 == 0 else qB

            @pl.loop(0, 64)
            def _(r):
                for hh in range(H):
                    buf[r, pl.ds(hh * DK, DK)] = acc[hh][
                        pl.ds((64 * m + r) * 16, 16)]

            pltpu.sync_copy(buf, wv_out.at[pl.ds(lo + 64 * m, 64)])

        pltpu.sync_copy(z_acc.at[pl.ds(0, RNG * 16)],
                        z_out.at[pl.ds(lo * 16, RNG * 16)])

    return k(q, kv, pck, counts)


_BQ = 2560


def _qkv_body(h_ref, qw, qb, kw, kb, vw, vb, q_ref, kv_ref):
    hb = h_ref[...]
    q_ref[...] = jnp.dot(hb, qw[...], preferred_element_type=jnp.float32) + qb[...]
    xk = jnp.dot(hb, kw[...], preferred_element_type=jnp.float32) + kb[...]
    xv = jnp.dot(hb, vw[...], preferred_element_type=jnp.float32) + vb[...]
    kv_ref[...] = jnp.concatenate([xk, xv], axis=1)


def _qkv(h, qw, qb, kw, kb, vw, vb):
    row = pl.BlockSpec((_BQ, D), lambda i: (i, 0))
    wspec = pl.BlockSpec((D, D), lambda i: (0, 0))
    bspec = pl.BlockSpec((1, D), lambda i: (0, 0))
    return pl.pallas_call(
        _qkv_body,
        grid=(NP // _BQ,),
        in_specs=[row, wspec, bspec, wspec, bspec, wspec, bspec],
        out_specs=[row, pl.BlockSpec((_BQ, 2 * D), lambda i: (i, 0))],
        out_shape=[jax.ShapeDtypeStruct((NP, D), jnp.float32),
                   jax.ShapeDtypeStruct((NP, 2 * D), jnp.float32)],
    )(h, qw, qb.reshape(1, D), kw, kb.reshape(1, D), vw, vb.reshape(1, D))


def _ln_blk(x, s, b):
    mu = jnp.mean(x, axis=-1, keepdims=True)
    d = x - mu
    var = jnp.mean(d * d, axis=-1, keepdims=True)
    return d * jax.lax.rsqrt(var + 1e-5) * s + b


def _post_body(hin_ref, wv_ref, z_ref, ow, ob, f1w, f1b, f2w, f2b,
               l1s, l1b, l2s, l2b, out_ref):
    wv = wv_ref[...]
    z = z_ref[...]
    hsel = (lax.broadcasted_iota(jnp.int32, (16, D), 1) // DK
            == lax.broadcasted_iota(jnp.int32, (16, D), 0)).astype(jnp.float32)
    zexp = jnp.dot(z, hsel, preferred_element_type=jnp.float32)
    attn = wv / jnp.where(zexp == 0.0, 1.0, zexp)
    h1 = hin_ref[...] + jnp.dot(attn, ow[...],
                                preferred_element_type=jnp.float32) + ob[...]
    h1 = _ln_blk(h1, l1s[...], l1b[...])
    f = jnp.maximum(jnp.dot(h1, f1w[...],
                            preferred_element_type=jnp.float32) + f1b[...], 0.0)
    h2 = h1 + jnp.dot(f, f2w[...], preferred_element_type=jnp.float32) + f2b[...]
    out_ref[...] = _ln_blk(h2, l2s[...], l2b[...])


def _post(hin, wv, z, ow, ob, f1w, f1b, f2w, f2b, l1s, l1b, l2s, l2b):
    row = pl.BlockSpec((_BQ, D), lambda i: (i, 0))
    zspec = pl.BlockSpec((_BQ, 16), lambda i: (i, 0))
    bspec = pl.BlockSpec((1, D), lambda i: (0, 0))
    return pl.pallas_call(
        _post_body,
        grid=(NP // _BQ,),
        in_specs=[row, row, zspec,
                  pl.BlockSpec((D, D), lambda i: (0, 0)), bspec,
                  pl.BlockSpec((D, 2 * D), lambda i: (0, 0)),
                  pl.BlockSpec((1, 2 * D), lambda i: (0, 0)),
                  pl.BlockSpec((2 * D, D), lambda i: (0, 0)), bspec,
                  bspec, bspec, bspec, bspec],
        out_specs=row,
        out_shape=jax.ShapeDtypeStruct((NP, D), jnp.float32),
    )(hin, wv, z, ow, ob.reshape(1, D), f1w, f1b.reshape(1, 2 * D), f2w,
      f2b.reshape(1, D), l1s.reshape(1, D), l1b.reshape(1, D),
      l2s.reshape(1, D), l2b.reshape(1, D))


def kernel(x, edge_index, QW, Qb, KW, Kb, VW, Vb, OW, Ob, F1W, F1b, F2W, F2b,
           LN1s, LN1b, LN2s, LN2b):
    src = edge_index[0].astype(jnp.int32)
    dst = edge_index[1].astype(jnp.int32)
    pck, counts = _bucket(src, dst)
    h = jnp.pad(x, ((0, NP - N), (0, 0)))
    for l in range(2):
        q, kv = _qkv(h, QW[l], Qb[l], KW[l], Kb[l], VW[l], Vb[l])
        wv, z = _edge(q, kv, pck, counts)
        z = z.reshape(NP, 16)
        h = _post(h, wv, z, OW[l], Ob[l], F1W[l], F1b[l], F2W[l], F2b[l],
                  LN1s[l], LN1b[l], LN2s[l], LN2b[l])
    return h[:N]
